# baseline (speedup 1.0000x reference)
import functools

import jax
import jax.numpy as jnp
from jax import lax
from jax.experimental import pallas as pl
from jax.experimental.pallas import tpu as pltpu
from jax.experimental.pallas import tpu_sc as plsc

_LANES = 128
_SUB = 5
_LK = 40
_NBLK = _SUB * _LANES // 16


def _make_sc_embed(B, L, NC, NS):
    NW = NC * NS
    Dp = _SUB * _LANES
    cols_per_tile = B // NW
    pairs_per_tile = cols_per_tile * L
    n_chunks = pairs_per_tile // _LK
    cpc = L // _LK

    mesh = plsc.VectorSubcoreMesh(core_axis_name="c", subcore_axis_name="s")

    @functools.partial(
        pl.kernel,
        mesh=mesh,
        out_type=jax.ShapeDtypeStruct((B, Dp), jnp.float32),
        scratch_types=[
            pltpu.VMEM((n_chunks, _LK), jnp.int32),
            pltpu.VMEM((_LK, Dp), jnp.float32),
            pltpu.VMEM((_LK, Dp), jnp.float32),
            pltpu.VMEM((cols_per_tile, Dp), jnp.float32),
            pltpu.SemaphoreType.DMA,
            pltpu.SemaphoreType.DMA,
        ],
    )
    def sc_embed(idx_hbm, tab_hbm, ep_hbm,
                 idx_v, buf_a, buf_b, stage, gs_a, gs_b):
        c = lax.axis_index("c")
        s = lax.axis_index("s")
        w = c * NS + s
        out_base = c * (B // NC) + s * cols_per_tile

        pltpu.sync_copy(idx_hbm.at[w], idx_v)

        def accumulate(buf, acc):
            def lk_body(lk, a):
                a = list(a)
                for blk in range(_NBLK):
                    a[blk] = a[blk] + buf[lk, pl.ds(blk * 16, 16)]
                return tuple(a)
            return lax.fori_loop(0, _LK, lk_body, acc)

        zeros_acc = tuple(
            jnp.zeros((16,), jnp.float32) for _ in range(_NBLK))

        pltpu.async_copy(tab_hbm.at[idx_v.at[0]], buf_a, gs_a)
        pltpu.async_copy(tab_hbm.at[idx_v.at[1]], buf_b, gs_b)

        def pair_body(cp, acc):
            base = 2 * cpc * cp
            for j in range(2 * cpc):
                buf, sem = (buf_a, gs_a) if j % 2 == 0 else (buf_b, gs_b)
                g = base + j
                pltpu.make_async_copy(tab_hbm.at[idx_v.at[g]], buf, sem).wait()
                acc = accumulate(buf, acc)

                @pl.when(g + 2 < n_chunks)
                def _():
                    pltpu.async_copy(tab_hbm.at[idx_v.at[g + 2]], buf, sem)

                if j % cpc == cpc - 1:
                    col = 2 * cp + j // cpc
                    for blk in range(_NBLK):
                        stage[col, pl.ds(blk * 16, 16)] = acc[blk]
                    acc = zeros_acc
            return acc

        lax.fori_loop(0, cols_per_tile // 2, pair_body, zeros_acc)

        pltpu.sync_copy(stage, ep_hbm.at[pl.ds(out_base, cols_per_tile)])

    return sc_embed, cols_per_tile, pairs_per_tile, n_chunks


def _mlp_body(ep_ref, wx_ref, b1_ref, w2_ref, b2_ref, out_ref):
    a = lax.dot_general(ep_ref[...], wx_ref[...], (((1,), (0,)), ((), ())),
                        preferred_element_type=jnp.float32)
    h = jnp.maximum(a + b1_ref[...][None, :], 0.0)
    out_ref[...] = jnp.sum(h * w2_ref[...], axis=1) + b2_ref[...]


def kernel(input, lut, static_lut, W1, b1, W2, b2):
    L, B = input.shape
    V, D = lut.shape
    Dp = _SUB * _LANES
    info = plsc.get_sparse_core_info()
    NC, NS = info.num_cores, info.num_subcores
    NW = NC * NS

    sc_embed, cols_per_tile, pairs_per_tile, n_chunks = _make_sc_embed(
        B, L, NC, NS)

    fused = (jnp.pad(lut, ((0, 0), (0, Dp - D)))
             + jnp.pad(static_lut, ((0, 0), (D, Dp - 2 * D))))

    idx = input.T.astype(jnp.int32).reshape(NW, n_chunks, _LK)

    ep = sc_embed(idx, fused)

    Wx = jnp.concatenate(
        [W1.T, jnp.zeros((Dp - 2 * D, 2 * D), jnp.float32)], axis=0)
    out = pl.pallas_call(
        _mlp_body,
        out_shape=jax.ShapeDtypeStruct((B,), jnp.float32),
    )(ep, Wx, b1, W2, b2)
    return out
